# baseline (speedup 1.0000x reference)
import functools

import jax
import jax.numpy as jnp
from jax import lax
from jax.experimental import pallas as pl
from jax.experimental.pallas import tpu as pltpu
from jax.experimental.pallas import tpu_sc as plsc

F = 26
K = 16
C = 16
CF = C * F


TR_BW = 8192


def _tr_kernel(in_ref, out_ref, scr):
    scr[...] = in_ref[...].T
    out_ref[...] = jnp.concatenate(
        [scr[pl.ds(r, TR_BW // 8, 8), :] for r in range(8)], axis=1)


def _row_major_table(emb_table):
    M = emb_table.shape[0]
    emb_t = jnp.swapaxes(emb_table, 0, 1)
    packed = pl.pallas_call(
        _tr_kernel,
        grid=(pl.cdiv(M, TR_BW),),
        in_specs=[pl.BlockSpec((16, TR_BW), lambda j: (0, j))],
        out_specs=pl.BlockSpec((TR_BW // 8, 128), lambda j: (j, 0)),
        out_shape=jax.ShapeDtypeStruct((M // 8, 128), jnp.float32),
        scratch_shapes=[pltpu.VMEM((TR_BW, 16), jnp.float32)],
    )(emb_t)
    return packed.reshape(M, 16)


def _permute(v, idx):
    dnums = lax.GatherDimensionNumbers(
        offset_dims=(), collapsed_slice_dims=(0,), start_index_map=(0,))
    return lax.gather(v, idx[:, None], dnums, slice_sizes=(1,),
                      mode=lax.GatherScatterMode.PROMISE_IN_BOUNDS)


def _fm_kernel(feat_hbm, emb_hbm, bias_hbm, out_hbm,
               idx0, idx1, rows0, rows1, bv0, bv1, out_v,
               se0, sb0, se1, sb1, si0, si1,
               *, nw, nc, rows_per_w, n_chunks):
    wid = lax.axis_index("s") * nc + lax.axis_index("c")
    w_base = wid * rows_per_w

    bv0[pl.ds(CF, 16)] = jnp.zeros((16,), jnp.float32)
    bv1[pl.ds(CF, 16)] = jnp.zeros((16,), jnp.float32)

    def idx_cp(c, iv, sem):
        fbase = (w_base + c * C) * F
        return pltpu.make_async_copy(feat_hbm.at[pl.ds(fbase, CF)], iv, sem)

    def e_cp(iv, rv, sem):
        return pltpu.make_async_copy(emb_hbm.at[iv], rv, sem)

    def b_cp(iv, bv, sem):
        return pltpu.make_async_copy(bias_hbm.at[iv], bv.at[pl.ds(0, CF)], sem)

    def compute_chunk(c, rows_v, bvals_v):
        lane = jnp.arange(16, dtype=jnp.int32)
        bmask = jnp.where(lane < (F - 16), jnp.float32(1.0), jnp.float32(0.0))

        def merge(x, y, d):
            t = x + _permute(x, lane ^ d)
            u = y + _permute(y, lane ^ d)
            return jnp.where((lane & d) == 0, t, u)

        def rev4(m):
            return int("{:04b}".format(m)[::-1], 2)

        for g in range(C // 16):
            tots = [None] * 16
            for q in range(4):
                rs = [g * 16 + q * 4 + i for i in range(4)]
                e0 = [rows_v[r * F] for r in rs]
                s = e0
                sq = [e * e for e in e0]
                for f in range(1, F):
                    es = [rows_v[r * F + f] for r in rs]
                    s = [a + e for a, e in zip(s, es)]
                    sq = [a + e * e for a, e in zip(sq, es)]
                for i, r in enumerate(rs):
                    v = 0.5 * (s[i] * s[i] - sq[i])
                    b1 = bvals_v[pl.ds(r * F, 16)]
                    b2 = bvals_v[pl.ds(r * F + 16, 16)] * bmask
                    tots[q * 4 + i] = v + b1 + b2
            vs = [tots[rev4(m)] for m in range(16)]
            for d in (8, 4, 2, 1):
                vs = [merge(vs[2 * i], vs[2 * i + 1], d)
                      for i in range(len(vs) // 2)]
            out_v[pl.ds(g * 16, 16)] = vs[0]

        pltpu.sync_copy(out_v, out_hbm.at[pl.ds(w_base + c * C, C)])

    def wrap(c):
        return jnp.where(c >= n_chunks, c - n_chunks, c)

    cp = idx_cp(0, idx0, si0)
    cp.start()
    cp.wait()
    e_cp(idx0, rows0, se0).start()
    b_cp(idx0, bv0, sb0).start()
    idx_cp(1, idx1, si1).start()

    def body(i, carry):
        c0 = 2 * i
        idx_cp(0, idx1, si1).wait()
        e_cp(idx1, rows1, se1).start()
        b_cp(idx1, bv1, sb1).start()

        e_cp(idx0, rows0, se0).wait()
        b_cp(idx0, bv0, sb0).wait()
        compute_chunk(c0, rows0, bv0)

        idx_cp(wrap(c0 + 2), idx0, si0).start()

        e_cp(idx1, rows1, se1).wait()
        b_cp(idx1, bv1, sb1).wait()
        compute_chunk(c0 + 1, rows1, bv1)

        idx_cp(0, idx0, si0).wait()
        e_cp(idx0, rows0, se0).start()
        b_cp(idx0, bv0, sb0).start()
        idx_cp(wrap(c0 + 3), idx1, si1).start()
        return carry

    lax.fori_loop(0, n_chunks // 2, body, 0)

    e_cp(idx0, rows0, se0).wait()
    b_cp(idx0, bv0, sb0).wait()
    idx_cp(0, idx1, si1).wait()


def kernel(features, labels, emb_table, bias_table, bias):
    B = features.shape[0]
    info = plsc.get_sparse_core_info()
    nc, ns = info.num_cores, info.num_subcores
    nw = nc * ns
    rows_per_w = B // nw
    n_chunks = rows_per_w // C

    feat_flat = features.reshape(-1).astype(jnp.int32)
    bias_flat = bias_table.reshape(-1)

    mesh = plsc.VectorSubcoreMesh(core_axis_name="c", subcore_axis_name="s")
    fm = pl.kernel(
        functools.partial(_fm_kernel, nw=nw, nc=nc,
                          rows_per_w=rows_per_w, n_chunks=n_chunks),
        mesh=mesh,
        compiler_params=pltpu.CompilerParams(use_tc_tiling_on_sc=False),
        out_type=jax.ShapeDtypeStruct((B,), jnp.float32),
        scratch_types=[
            pltpu.VMEM((CF,), jnp.int32),
            pltpu.VMEM((CF,), jnp.int32),
            pltpu.VMEM((CF, K), jnp.float32),
            pltpu.VMEM((CF, K), jnp.float32),
            pltpu.VMEM((CF + 16,), jnp.float32),
            pltpu.VMEM((CF + 16,), jnp.float32),
            pltpu.VMEM((C,), jnp.float32),
            pltpu.SemaphoreType.DMA,
            pltpu.SemaphoreType.DMA,
            pltpu.SemaphoreType.DMA,
            pltpu.SemaphoreType.DMA,
            pltpu.SemaphoreType.DMA,
            pltpu.SemaphoreType.DMA,
        ],
    )
    out = fm(feat_flat, _row_major_table(emb_table), bias_flat)
    return out.reshape(B, 1) + bias
